# baseline (speedup 1.0000x reference)
import jax
import jax.numpy as jnp
from jax import lax
from jax.experimental import pallas as pl
from jax.experimental.pallas import tpu as pltpu
from jax.experimental.pallas import tpu_sc as plsc

_VOCAB = 100000
_DIM = 64
_NC = 2
_NS = 16
_NW = _NC * _NS

_B = 4096
_J = 200
_IBLK = _B // _NW
_NCHUNK = _J
_NBUF = 4
_A = 2


def _gather_body(idx_hbm, tab_hbm, out_hbm, *scratch):
    idxs = scratch[0:_NBUF]
    rows = scratch[_NBUF:2 * _NBUF]
    si = scratch[2 * _NBUF:3 * _NBUF]
    sg = scratch[3 * _NBUF:4 * _NBUF]
    sw = scratch[4 * _NBUF:5 * _NBUF]

    wid = lax.axis_index("s") * _NC + lax.axis_index("c")
    i0 = wid * _IBLK

    def fire_idx(c, b):
        pltpu.async_copy(idx_hbm.at[c, pl.ds(i0, _IBLK)], idxs[b], si[b])

    def wait_idx(b):
        pltpu.make_async_copy(idx_hbm.at[0, pl.ds(i0, _IBLK)],
                              idxs[b], si[b]).wait()

    def fire_gather(b):
        pltpu.async_copy(tab_hbm.at[idxs[b]], rows[b], sg[b])

    def wait_gather(b):
        pltpu.make_async_copy(tab_hbm.at[idxs[b]], rows[b], sg[b]).wait()

    def fire_wb(c, b):
        pltpu.async_copy(rows[b], out_hbm.at[c, pl.ds(i0, _IBLK)], sw[b])

    def wait_wb(b):
        pltpu.make_async_copy(rows[b], out_hbm.at[0, pl.ds(i0, _IBLK)],
                              sw[b]).wait()

    for c in range(_A + 1):
        fire_idx(c, c % _NBUF)
    for c in range(_A):
        wait_idx(c % _NBUF)
        fire_gather(c % _NBUF)

    def group(g, carry):
        for b in range(_NBUF):
            k = g * _NBUF + b
            ba = (b + _A) % _NBUF
            bn = (b + _A + 1) % _NBUF

            @pl.when(k + _A < _NCHUNK)
            def _():
                wait_idx(ba)

                @pl.when(k + _A >= _NBUF)
                def _():
                    wait_wb(ba)

                fire_gather(ba)

            wait_gather(b)
            fire_wb(k, b)

            @pl.when(k + _A + 1 < _NCHUNK)
            def _():
                fire_idx(k + _A + 1, bn)
        return carry

    lax.fori_loop(0, _NCHUNK // _NBUF, group, 0)

    for b in range(_NBUF):
        wait_wb(b)


_mesh = plsc.VectorSubcoreMesh(core_axis_name="c", subcore_axis_name="s")


def kernel(tensor, table_fix, table_v):
    idx = jnp.swapaxes(tensor, 0, 1).astype(jnp.int32)
    tab = table_fix.at[_VOCAB - 3:].add(table_v)
    call = pl.kernel(
        _gather_body,
        out_type=jax.ShapeDtypeStruct((_J, _B, _DIM), jnp.float32),
        mesh=_mesh,
        scratch_types=(
            [pltpu.VMEM((_IBLK,), jnp.int32) for _ in range(_NBUF)]
            + [pltpu.VMEM((_IBLK, _DIM), jnp.float32) for _ in range(_NBUF)]
            + [pltpu.SemaphoreType.DMA for _ in range(3 * _NBUF)]
        ),
        compiler_params=pltpu.CompilerParams(use_tc_tiling_on_sc=False),
    )
    return call(idx, tab)
